# Initial kernel scaffold; baseline (speedup 1.0000x reference)
import jax
import jax.numpy as jnp
from jax.experimental import pallas as pl


def kernel(user_latent_value, item_latent_value, user_idx):
    raise NotImplementedError("write your pallas kernel here")


# speedup vs baseline: 1.1134x; 1.1134x over previous
import functools

import jax
import jax.numpy as jnp
from jax import lax
from jax.experimental import pallas as pl
from jax.experimental.pallas import tpu as pltpu
from jax.experimental.pallas import tpu_sc as plsc


def _log_softmax_table_kernel(u_ref, it_ref, out_ref):
    u = u_ref[...]
    it = it_ref[...]
    util = lax.dot_general(u, it, (((1,), (1,)), ((), ())),
                           preferred_element_type=jnp.float32)
    m = jnp.max(util, axis=-1, keepdims=True)
    e = jnp.exp(util - m)
    lse = m + jnp.log(jnp.sum(e, axis=-1, keepdims=True))
    out_ref[...] = util - lse


@functools.cache
def _make_gather(U, I, B):
    info = plsc.get_sparse_core_info()
    NC, NS = info.num_cores, info.num_subcores
    NW = NC * NS
    assert B % NW == 0
    b_per_w = B // NW
    C = 64
    while b_per_w % C:
        C //= 2
    n_chunks = b_per_w // C
    mesh = plsc.VectorSubcoreMesh(core_axis_name="c", subcore_axis_name="s")

    @functools.partial(
        pl.kernel, mesh=mesh,
        out_type=jax.ShapeDtypeStruct((B, I), jnp.float32),
        compiler_params=pltpu.CompilerParams(use_tc_tiling_on_sc=False),
        scratch_types=[
            pltpu.VMEM((b_per_w,), jnp.int32),
            pltpu.VMEM((C, I), jnp.float32),
            pltpu.VMEM((C, I), jnp.float32),
            pltpu.SemaphoreType.DMA,
            pltpu.SemaphoreType.DMA,
        ],
    )
    def gather(table_hbm, idx_hbm, out_hbm, idx_v, buf0, buf1, sem0, sem1):
        wid = lax.axis_index("s") * NC + lax.axis_index("c")
        base = wid * b_per_w
        pltpu.sync_copy(idx_hbm.at[pl.ds(base, b_per_w)], idx_v)
        bufs = (buf0, buf1)
        sems = (sem0, sem1)
        copies = [None, None]
        copies[0] = pltpu.async_copy(
            table_hbm.at[idx_v.at[pl.ds(0, C)]], bufs[0], sems[0])
        for c in range(n_chunks):
            cur, nxt = c % 2, (c + 1) % 2
            if c + 1 < n_chunks:
                copies[nxt] = pltpu.async_copy(
                    table_hbm.at[idx_v.at[pl.ds((c + 1) * C, C)]],
                    bufs[nxt], sems[nxt])
            copies[cur].wait()
            pltpu.sync_copy(bufs[cur], out_hbm.at[pl.ds(base + c * C, C)])

    return gather


def kernel(user_latent_value, item_latent_value, user_idx):
    S, U, D = user_latent_value.shape
    I = item_latent_value.shape[1]
    B = user_idx.shape[0]
    u2 = user_latent_value.reshape(U, D)
    it2 = item_latent_value.reshape(I, D)
    table = pl.pallas_call(
        _log_softmax_table_kernel,
        out_shape=jax.ShapeDtypeStruct((U, I), jnp.float32),
    )(u2, it2)
    out = _make_gather(U, I, B)(table, user_idx.astype(jnp.int32))
    return out.reshape(S, B, I)


# speedup vs baseline: 1.7424x; 1.5649x over previous
import functools

import jax
import jax.numpy as jnp
from jax import lax
from jax.experimental import pallas as pl
from jax.experimental.pallas import tpu as pltpu
from jax.experimental.pallas import tpu_sc as plsc

_LANES = 128


def _log_softmax_table_kernel(nitems, u_ref, it_ref, out_ref):
    u = u_ref[...]
    it = it_ref[...]
    util = lax.dot_general(u, it, (((1,), (1,)), ((), ())),
                           preferred_element_type=jnp.float32)
    U, Ipad = util.shape
    valid = lax.broadcasted_iota(jnp.int32, (1, Ipad), 1) < nitems
    util = jnp.where(valid, util, -jnp.inf)
    m = jnp.max(util, axis=-1, keepdims=True)
    e = jnp.exp(util - m)
    lse = m + jnp.log(jnp.sum(e, axis=-1, keepdims=True))
    logp = jnp.where(valid, util - lse, 0.0)
    for j in range(Ipad // _LANES):
        out_ref[pl.ds(j * U, U), :] = logp[:, j * _LANES:(j + 1) * _LANES]


@functools.cache
def _make_gather(U, I, B):
    info = plsc.get_sparse_core_info()
    NC, NS = info.num_cores, info.num_subcores
    NW = NC * NS
    assert B % NW == 0
    b_per_w = B // NW
    C = 32
    while b_per_w % C:
        C //= 2
    n_chunks = b_per_w // C
    assert n_chunks % 2 == 0
    NT = (I + _LANES - 1) // _LANES
    TAIL = I - (NT - 1) * _LANES
    mesh = plsc.VectorSubcoreMesh(core_axis_name="c", subcore_axis_name="s")

    @functools.partial(
        pl.kernel, mesh=mesh,
        out_type=jax.ShapeDtypeStruct((B, I), jnp.float32),
        scratch_types=[
            pltpu.VMEM((b_per_w,), jnp.int32),
            pltpu.VMEM((NT, C), jnp.int32),
            pltpu.VMEM((NT, C), jnp.int32),
            pltpu.VMEM((NT, C, _LANES), jnp.float32),
            pltpu.VMEM((NT, C, _LANES), jnp.float32),
            pltpu.VMEM((C, TAIL), jnp.float32),
            pltpu.VMEM((C, TAIL), jnp.float32),
            pltpu.SemaphoreType.DMA,
            pltpu.SemaphoreType.DMA,
        ],
    )
    def gather(table_hbm, idx_hbm, out_hbm,
               idx_v, midxA, midxB, tbufA, tbufB, t7A, t7B, semA, semB):
        wid = lax.axis_index("s") * NC + lax.axis_index("c")
        base = wid * b_per_w
        pltpu.sync_copy(idx_hbm.at[pl.ds(base, b_per_w)], idx_v)

        def build(midx, c):
            for k in range(C // 16):
                v = idx_v[pl.ds(c * C + 16 * k, 16)]
                for j in range(NT):
                    midx[j, pl.ds(16 * k, 16)] = v + j * U

        def fire(midx, tbuf, sem):
            return [pltpu.async_copy(table_hbm.at[midx.at[j]], tbuf.at[j], sem)
                    for j in range(NT)]

        def drain(copies, tbuf, t7, c):
            for cp in copies:
                cp.wait()
            row0 = pl.multiple_of(base + c * C, C)
            for j in range(NT - 1):
                pltpu.sync_copy(
                    tbuf.at[j],
                    out_hbm.at[pl.ds(row0, C), pl.ds(j * _LANES, _LANES)])
            nvec = TAIL // 16
            offs = [16 * k for k in range(nvec)] + ([TAIL - 16] if TAIL % 16 else [])

            def row_copy(r, carry):
                for off in offs:
                    t7[r, pl.ds(off, 16)] = tbuf[NT - 1, r, pl.ds(off, 16)]
                return carry

            lax.fori_loop(0, C, row_copy, 0)
            pltpu.sync_copy(
                t7, out_hbm.at[pl.ds(row0, C), pl.ds((NT - 1) * _LANES, TAIL)])

        sets = ((midxA, tbufA, t7A, semA), (midxB, tbufB, t7B, semB))
        build(midxA, 0)
        pend = [fire(midxA, tbufA, semA), None]
        for c in range(n_chunks):
            cur, nxt = c % 2, (c + 1) % 2
            if c + 1 < n_chunks:
                build(sets[nxt][0], c + 1)
                pend[nxt] = fire(sets[nxt][0], sets[nxt][1], sets[nxt][3])
            drain(pend[cur], sets[cur][1], sets[cur][2], c)

    return gather


def kernel(user_latent_value, item_latent_value, user_idx):
    S, U, D = user_latent_value.shape
    I = item_latent_value.shape[1]
    B = user_idx.shape[0]
    Ipad = (I + _LANES - 1) // _LANES * _LANES
    u2 = user_latent_value.reshape(U, D)
    it2 = item_latent_value.reshape(I, D)
    it2 = jnp.pad(it2, ((0, Ipad - I), (0, 0)))
    table = pl.pallas_call(
        functools.partial(_log_softmax_table_kernel, I),
        out_shape=jax.ShapeDtypeStruct(((Ipad // _LANES) * U, _LANES),
                                       jnp.float32),
    )(u2, it2)
    out = _make_gather(U, I, B)(table, user_idx.astype(jnp.int32))
    return out.reshape(S, B, I)
